# baseline (speedup 1.0000x reference)
import functools

import jax
import jax.numpy as jnp
from jax import lax
from jax.experimental import pallas as pl
from jax.experimental.pallas import tpu as pltpu
from jax.experimental.pallas import tpu_sc as plsc

N_NODES = 10000
N_EDGES = 320000
D = 128
OUT_DIM = 64

NC, NS = 2, 16
NW = NC * NS
E_PER_W = N_EDGES // NW
CHUNK = 80
NCHUNK = E_PER_W // CHUNK
N_PAD = 10240
H_PER_TILE = N_PAD // NS

GCHUNK = 80
GNCHUNK = 125
WIN = 5
NWIN = GNCHUNK // WIN


def _mesh():
    return plsc.VectorSubcoreMesh(
        core_axis_name="c", subcore_axis_name="s", num_cores=NC, num_subcores=NS
    )


def _sc_hist(dst3):
    @functools.partial(
        pl.kernel,
        mesh=_mesh(),
        out_type=jax.ShapeDtypeStruct((NC, 1, N_PAD), jnp.float32),
        scratch_types=[
            pltpu.VMEM((NCHUNK, CHUNK), jnp.int32),
            pltpu.VMEM((CHUNK,), jnp.float32),
            pltpu.VMEM((H_PER_TILE,), jnp.float32),
            pltpu.VMEM_SHARED((N_PAD,), jnp.float32),
        ],
    )
    def k(dst_hbm, out_hbm, dst_v, ones_v, zbuf_v, hist_sh):
        c = lax.axis_index("c")
        s = lax.axis_index("s")
        w = c * NS + s
        zeros16 = jnp.zeros((16,), jnp.float32)
        ones16 = jnp.ones((16,), jnp.float32)

        def zero_body(i, carry):
            zbuf_v[pl.ds(i * 16, 16)] = zeros16
            return carry

        lax.fori_loop(0, H_PER_TILE // 16, zero_body, 0)
        for q in range(CHUNK // 16):
            ones_v[pl.ds(q * 16, 16)] = ones16
        pltpu.sync_copy(zbuf_v, hist_sh.at[pl.ds(s * H_PER_TILE, H_PER_TILE)])
        pltpu.sync_copy(dst_hbm.at[w], dst_v)
        plsc.subcore_barrier()

        def body(j, carry):
            pltpu.sync_copy(ones_v, hist_sh.at[dst_v.at[j]], add=True)
            return carry

        lax.fori_loop(0, NCHUNK, body, 0)
        plsc.subcore_barrier()
        pltpu.sync_copy(
            hist_sh.at[pl.ds(s * H_PER_TILE, H_PER_TILE)],
            out_hbm.at[c, 0, pl.ds(s * H_PER_TILE, H_PER_TILE)],
        )

    return k(dst3)


def _sc_scatter(hs, src3, dst3, zrows):
    @functools.partial(
        pl.kernel,
        mesh=_mesh(),
        out_type=jax.ShapeDtypeStruct((NC, N_PAD, D), jnp.float32),
        scratch_types=[
            pltpu.VMEM((3, WIN, GCHUNK), jnp.int32),
            pltpu.VMEM((3, WIN, GCHUNK), jnp.int32),
            pltpu.VMEM((4, GCHUNK, D), jnp.float32),
            pltpu.VMEM_SHARED((N_PAD, D), jnp.float32),
            pltpu.SemaphoreType.DMA((4,)),
            pltpu.SemaphoreType.DMA((4,)),
            pltpu.SemaphoreType.DMA((3,)),
        ],
    )
    def k(hs_hbm, src_hbm, dst_hbm, z_hbm, out_hbm, src_v, dst_v, rows_v, acc_sh, gsem, ssem, isem):
        c = lax.axis_index("c")
        s = lax.axis_index("s")
        w = c * NS + s
        zcopy = pltpu.async_copy(
            z_hbm.at[pl.ds(s * H_PER_TILE, H_PER_TILE)],
            acc_sh.at[pl.ds(s * H_PER_TILE, H_PER_TILE)],
            gsem.at[2],
        )
        pltpu.sync_copy(src_hbm.at[w, 0], src_v.at[0])
        pltpu.sync_copy(dst_hbm.at[w, 0], dst_v.at[0])
        pltpu.async_copy(src_hbm.at[w, 1], src_v.at[1], isem.at[1])
        pltpu.async_copy(dst_hbm.at[w, 1], dst_v.at[1], isem.at[1])
        zcopy.wait()
        plsc.subcore_barrier()

        pltpu.async_copy(hs_hbm.at[src_v.at[0, 0]], rows_v.at[0], gsem.at[0])
        pltpu.async_copy(hs_hbm.at[src_v.at[0, 1]], rows_v.at[1], gsem.at[1])

        def outer(o, carry):
            om = lax.rem(o, 3)
            om1 = lax.rem(o + 1, 3)
            om2 = lax.rem(o + 2, 3)
            for j in range(WIN):
                cidx = o * WIN + j
                b = lax.rem(cidx, 4)

                if j == 3:
                    @pl.when(o + 1 < NWIN)
                    def _():
                        pltpu.make_async_copy(
                            src_hbm.at[w, o + 1], src_v.at[om1], isem.at[om1]
                        ).wait()
                        pltpu.make_async_copy(
                            dst_hbm.at[w, o + 1], dst_v.at[om1], isem.at[om1]
                        ).wait()

                g = cidx + 2
                bg = lax.rem(g, 4)
                gidx = src_v.at[om, j + 2] if j < 3 else src_v.at[om1, j - 3]

                @pl.when(g < GNCHUNK)
                def _():
                    @pl.when(cidx >= 2)
                    def _():
                        pltpu.make_async_copy(
                            rows_v.at[bg], acc_sh.at[dst_v.at[om, 0]], ssem.at[bg]
                        ).wait()

                    pltpu.async_copy(hs_hbm.at[gidx], rows_v.at[bg], gsem.at[bg])

                pltpu.make_async_copy(
                    hs_hbm.at[src_v.at[om, j]], rows_v.at[b], gsem.at[b]
                ).wait()
                pltpu.async_copy(
                    rows_v.at[b], acc_sh.at[dst_v.at[om, j]], ssem.at[b], add=True
                )

                if j == 4:

                    @pl.when(o + 2 < NWIN)
                    def _():
                        pltpu.async_copy(src_hbm.at[w, o + 2], src_v.at[om2], isem.at[om2])
                        pltpu.async_copy(dst_hbm.at[w, o + 2], dst_v.at[om2], isem.at[om2])

            return carry

        lax.fori_loop(0, NWIN, outer, 0)
        for b in range(4):
            pltpu.make_async_copy(
                rows_v.at[b], acc_sh.at[dst_v.at[0, 0]], ssem.at[b]
            ).wait()
        plsc.subcore_barrier()
        pltpu.sync_copy(
            acc_sh.at[pl.ds(s * H_PER_TILE, H_PER_TILE)],
            out_hbm.at[c, pl.ds(s * H_PER_TILE, H_PER_TILE)],
        )

    return k(hs, src3, dst3, zrows)


_R = 2000


def _tc_matmul(x, W1):
    def body(x_ref, w_ref, out_ref):
        out_ref[...] = jnp.dot(x_ref[...], w_ref[...], preferred_element_type=jnp.float32)

    return pl.pallas_call(
        body,
        grid=(N_NODES // _R,),
        in_specs=[
            pl.BlockSpec((_R, D), lambda i: (i, 0)),
            pl.BlockSpec((D, D), lambda i: (0, 0)),
        ],
        out_specs=pl.BlockSpec((_R, D), lambda i: (i, 0)),
        out_shape=jax.ShapeDtypeStruct((N_NODES, D), jnp.float32),
    )(x, W1)


def _tc_scale(h1, hist):
    def body(h_ref, h0_ref, h1_ref, hs_ref, dinv_ref):
        deg = h0_ref[0] + h1_ref[0] + 1.0
        dinv = lax.rsqrt(deg)
        hs_ref[...] = h_ref[...] * dinv
        dinv_ref[...] = dinv

    return pl.pallas_call(
        body,
        grid=(N_NODES // _R,),
        in_specs=[
            pl.BlockSpec((_R, D), lambda i: (i, 0)),
            pl.BlockSpec((1, _R, 1), lambda i: (0, i, 0)),
            pl.BlockSpec((1, _R, 1), lambda i: (1, i, 0)),
        ],
        out_specs=[
            pl.BlockSpec((_R, D), lambda i: (i, 0)),
            pl.BlockSpec((_R, 1), lambda i: (i, 0)),
        ],
        out_shape=[
            jax.ShapeDtypeStruct((N_NODES, D), jnp.float32),
            jax.ShapeDtypeStruct((N_NODES, 1), jnp.float32),
        ],
    )(h1, hist, hist)


def _tc_mid(agg, hs_prev, dinv, b, W):
    def body(a0_ref, a1_ref, hsp_ref, dinv_ref, b_ref, w_ref, out_ref):
        dinv = dinv_ref[...]
        pre = (a0_ref[0] + a1_ref[0] + hsp_ref[...]) * dinv + b_ref[...]
        h = jnp.maximum(pre, 0.0)
        out_ref[...] = jnp.dot(h, w_ref[...], preferred_element_type=jnp.float32) * dinv

    return pl.pallas_call(
        body,
        grid=(N_NODES // _R,),
        in_specs=[
            pl.BlockSpec((1, _R, D), lambda i: (0, i, 0)),
            pl.BlockSpec((1, _R, D), lambda i: (1, i, 0)),
            pl.BlockSpec((_R, D), lambda i: (i, 0)),
            pl.BlockSpec((_R, 1), lambda i: (i, 0)),
            pl.BlockSpec((1, D), lambda i: (0, 0)),
            pl.BlockSpec((D, D), lambda i: (0, 0)),
        ],
        out_specs=pl.BlockSpec((_R, D), lambda i: (i, 0)),
        out_shape=jax.ShapeDtypeStruct((N_NODES, D), jnp.float32),
    )(agg, agg, hs_prev, dinv, b, W)


def _tc_last(agg, hs_prev, dinv, b, Wh, bh):
    def body(a0_ref, a1_ref, hsp_ref, dinv_ref, b_ref, wh_ref, bh_ref, out_ref):
        pre = (a0_ref[0] + a1_ref[0] + hsp_ref[...]) * dinv_ref[...] + b_ref[...]
        h = jnp.maximum(pre, 0.0)
        out_ref[...] = (
            jnp.dot(h, wh_ref[...], preferred_element_type=jnp.float32) + bh_ref[...]
        )

    return pl.pallas_call(
        body,
        grid=(N_NODES // _R,),
        in_specs=[
            pl.BlockSpec((1, _R, D), lambda i: (0, i, 0)),
            pl.BlockSpec((1, _R, D), lambda i: (1, i, 0)),
            pl.BlockSpec((_R, D), lambda i: (i, 0)),
            pl.BlockSpec((_R, 1), lambda i: (i, 0)),
            pl.BlockSpec((1, D), lambda i: (0, 0)),
            pl.BlockSpec((D, OUT_DIM), lambda i: (0, 0)),
            pl.BlockSpec((1, OUT_DIM), lambda i: (0, 0)),
        ],
        out_specs=pl.BlockSpec((_R, OUT_DIM), lambda i: (i, 0)),
        out_shape=jax.ShapeDtypeStruct((N_NODES, OUT_DIM), jnp.float32),
    )(agg, agg, hs_prev, dinv, b, Wh, bh)


def kernel(x, edge_index, W1, b1, W2, b2, Wh, bh):
    ei = edge_index.astype(jnp.int32).reshape(2, NW, NWIN, WIN, GCHUNK)
    zrows = jnp.zeros((N_PAD, D), jnp.float32)

    src3 = ei[0]
    dst3 = ei[1]
    hist = _sc_hist(dst3.reshape(NW, NCHUNK, CHUNK))
    h1 = _tc_matmul(x, W1)

    hs1, dinv = _tc_scale(h1, hist.reshape(NC, N_PAD, 1))
    agg1 = _sc_scatter(hs1, src3, dst3, zrows)
    hs2 = _tc_mid(agg1, hs1, dinv, b1.reshape(1, D), W2)
    agg2 = _sc_scatter(hs2, src3, dst3, zrows)
    z = _tc_last(agg2, hs2, dinv, b2.reshape(1, D), Wh, bh.reshape(1, OUT_DIM))
    return z


# speedup vs baseline: 1.0322x; 1.0322x over previous
import functools

import jax
import jax.numpy as jnp
from jax import lax
from jax.experimental import pallas as pl
from jax.experimental.pallas import tpu as pltpu
from jax.experimental.pallas import tpu_sc as plsc

N_NODES = 10000
N_EDGES = 320000
D = 128
OUT_DIM = 64

NC, NS = 2, 16
NW = NC * NS
E_PER_W = N_EDGES // NW
CHUNK = 80
NCHUNK = E_PER_W // CHUNK
N_PAD = 10240
H_PER_TILE = N_PAD // NS

GCHUNK = 80
GNCHUNK = 125
WIN = 5
NWIN = GNCHUNK // WIN


def _mesh():
    return plsc.VectorSubcoreMesh(
        core_axis_name="c", subcore_axis_name="s", num_cores=NC, num_subcores=NS
    )


def _sc_hist(dst3):
    @functools.partial(
        pl.kernel,
        mesh=_mesh(),
        out_type=jax.ShapeDtypeStruct((NC, 1, N_PAD), jnp.float32),
        scratch_types=[
            pltpu.VMEM((NCHUNK, CHUNK), jnp.int32),
            pltpu.VMEM((CHUNK,), jnp.float32),
            pltpu.VMEM((H_PER_TILE,), jnp.float32),
            pltpu.VMEM_SHARED((N_PAD,), jnp.float32),
        ],
    )
    def k(dst_hbm, out_hbm, dst_v, ones_v, zbuf_v, hist_sh):
        c = lax.axis_index("c")
        s = lax.axis_index("s")
        w = c * NS + s
        zeros16 = jnp.zeros((16,), jnp.float32)
        ones16 = jnp.ones((16,), jnp.float32)

        def zero_body(i, carry):
            zbuf_v[pl.ds(i * 16, 16)] = zeros16
            return carry

        lax.fori_loop(0, H_PER_TILE // 16, zero_body, 0)
        for q in range(CHUNK // 16):
            ones_v[pl.ds(q * 16, 16)] = ones16
        pltpu.sync_copy(zbuf_v, hist_sh.at[pl.ds(s * H_PER_TILE, H_PER_TILE)])
        pltpu.sync_copy(dst_hbm.at[w], dst_v)
        plsc.subcore_barrier()

        def body(j, carry):
            pltpu.sync_copy(ones_v, hist_sh.at[dst_v.at[j]], add=True)
            return carry

        lax.fori_loop(0, NCHUNK, body, 0)
        plsc.subcore_barrier()
        pltpu.sync_copy(
            hist_sh.at[pl.ds(s * H_PER_TILE, H_PER_TILE)],
            out_hbm.at[c, 0, pl.ds(s * H_PER_TILE, H_PER_TILE)],
        )

    return k(dst3)


def _sc_scatter(hs, ei, zrows):
    @functools.partial(
        pl.kernel,
        mesh=_mesh(),
        out_type=jax.ShapeDtypeStruct((NC, N_PAD, D), jnp.float32),
        scratch_types=[
            pltpu.VMEM((3, WIN, GCHUNK), jnp.int32),
            pltpu.VMEM((3, WIN, GCHUNK), jnp.int32),
            pltpu.VMEM((4, GCHUNK, D), jnp.float32),
            pltpu.VMEM_SHARED((N_PAD, D), jnp.float32),
            pltpu.SemaphoreType.DMA((4,)),
            pltpu.SemaphoreType.DMA((4,)),
            pltpu.SemaphoreType.DMA((3,)),
        ],
    )
    def k(hs_hbm, ei_hbm, z_hbm, out_hbm, src_v, dst_v, rows_v, acc_sh, gsem, ssem, isem):
        src_hbm = ei_hbm.at[0]
        dst_hbm = ei_hbm.at[1]
        c = lax.axis_index("c")
        s = lax.axis_index("s")
        w = c * NS + s
        zcopy = pltpu.async_copy(
            z_hbm.at[pl.ds(s * H_PER_TILE, H_PER_TILE)],
            acc_sh.at[pl.ds(s * H_PER_TILE, H_PER_TILE)],
            gsem.at[2],
        )
        pltpu.sync_copy(src_hbm.at[w, 0], src_v.at[0])
        pltpu.sync_copy(dst_hbm.at[w, 0], dst_v.at[0])
        pltpu.async_copy(src_hbm.at[w, 1], src_v.at[1], isem.at[1])
        pltpu.async_copy(dst_hbm.at[w, 1], dst_v.at[1], isem.at[1])
        zcopy.wait()
        plsc.subcore_barrier()

        pltpu.async_copy(hs_hbm.at[src_v.at[0, 0]], rows_v.at[0], gsem.at[0])
        pltpu.async_copy(hs_hbm.at[src_v.at[0, 1]], rows_v.at[1], gsem.at[1])

        def outer(o, carry):
            om = lax.rem(o, 3)
            om1 = lax.rem(o + 1, 3)
            om2 = lax.rem(o + 2, 3)
            for j in range(WIN):
                cidx = o * WIN + j
                b = lax.rem(cidx, 4)

                if j == 3:
                    @pl.when(o + 1 < NWIN)
                    def _():
                        pltpu.make_async_copy(
                            src_hbm.at[w, o + 1], src_v.at[om1], isem.at[om1]
                        ).wait()
                        pltpu.make_async_copy(
                            dst_hbm.at[w, o + 1], dst_v.at[om1], isem.at[om1]
                        ).wait()

                g = cidx + 2
                bg = lax.rem(g, 4)
                gidx = src_v.at[om, j + 2] if j < 3 else src_v.at[om1, j - 3]

                @pl.when(g < GNCHUNK)
                def _():
                    @pl.when(cidx >= 2)
                    def _():
                        pltpu.make_async_copy(
                            rows_v.at[bg], acc_sh.at[dst_v.at[om, 0]], ssem.at[bg]
                        ).wait()

                    pltpu.async_copy(hs_hbm.at[gidx], rows_v.at[bg], gsem.at[bg])

                pltpu.make_async_copy(
                    hs_hbm.at[src_v.at[om, j]], rows_v.at[b], gsem.at[b]
                ).wait()
                pltpu.async_copy(
                    rows_v.at[b], acc_sh.at[dst_v.at[om, j]], ssem.at[b], add=True
                )

                if j == 4:

                    @pl.when(o + 2 < NWIN)
                    def _():
                        pltpu.async_copy(src_hbm.at[w, o + 2], src_v.at[om2], isem.at[om2])
                        pltpu.async_copy(dst_hbm.at[w, o + 2], dst_v.at[om2], isem.at[om2])

            return carry

        lax.fori_loop(0, NWIN, outer, 0)
        for b in range(4):
            pltpu.make_async_copy(
                rows_v.at[b], acc_sh.at[dst_v.at[0, 0]], ssem.at[b]
            ).wait()
        plsc.subcore_barrier()
        pltpu.sync_copy(
            acc_sh.at[pl.ds(s * H_PER_TILE, H_PER_TILE)],
            out_hbm.at[c, pl.ds(s * H_PER_TILE, H_PER_TILE)],
        )

    return k(hs, ei, zrows)


_R = 2000


def _tc_matmul(x, W1):
    def body(x_ref, w_ref, out_ref):
        out_ref[...] = jnp.dot(x_ref[...], w_ref[...], preferred_element_type=jnp.float32)

    return pl.pallas_call(
        body,
        grid=(N_NODES // _R,),
        in_specs=[
            pl.BlockSpec((_R, D), lambda i: (i, 0)),
            pl.BlockSpec((D, D), lambda i: (0, 0)),
        ],
        out_specs=pl.BlockSpec((_R, D), lambda i: (i, 0)),
        out_shape=jax.ShapeDtypeStruct((N_NODES, D), jnp.float32),
    )(x, W1)


def _tc_scale(h1, hist):
    def body(h_ref, h0_ref, h1_ref, hs_ref, dinv_ref):
        deg = h0_ref[0] + h1_ref[0] + 1.0
        dinv = lax.rsqrt(deg)
        hs_ref[...] = h_ref[...] * dinv
        dinv_ref[...] = dinv

    return pl.pallas_call(
        body,
        grid=(N_NODES // _R,),
        in_specs=[
            pl.BlockSpec((_R, D), lambda i: (i, 0)),
            pl.BlockSpec((1, _R, 1), lambda i: (0, i, 0)),
            pl.BlockSpec((1, _R, 1), lambda i: (1, i, 0)),
        ],
        out_specs=[
            pl.BlockSpec((_R, D), lambda i: (i, 0)),
            pl.BlockSpec((_R, 1), lambda i: (i, 0)),
        ],
        out_shape=[
            jax.ShapeDtypeStruct((N_NODES, D), jnp.float32),
            jax.ShapeDtypeStruct((N_NODES, 1), jnp.float32),
        ],
    )(h1, hist, hist)


def _tc_mid(agg, hs_prev, dinv, b, W):
    def body(a0_ref, a1_ref, hsp_ref, dinv_ref, b_ref, w_ref, out_ref):
        dinv = dinv_ref[...]
        pre = (a0_ref[0] + a1_ref[0] + hsp_ref[...]) * dinv + b_ref[...]
        h = jnp.maximum(pre, 0.0)
        out_ref[...] = jnp.dot(h, w_ref[...], preferred_element_type=jnp.float32) * dinv

    return pl.pallas_call(
        body,
        grid=(N_NODES // _R,),
        in_specs=[
            pl.BlockSpec((1, _R, D), lambda i: (0, i, 0)),
            pl.BlockSpec((1, _R, D), lambda i: (1, i, 0)),
            pl.BlockSpec((_R, D), lambda i: (i, 0)),
            pl.BlockSpec((_R, 1), lambda i: (i, 0)),
            pl.BlockSpec((1, D), lambda i: (0, 0)),
            pl.BlockSpec((D, D), lambda i: (0, 0)),
        ],
        out_specs=pl.BlockSpec((_R, D), lambda i: (i, 0)),
        out_shape=jax.ShapeDtypeStruct((N_NODES, D), jnp.float32),
    )(agg, agg, hs_prev, dinv, b, W)


def _tc_last(agg, hs_prev, dinv, b, Wh, bh):
    def body(a0_ref, a1_ref, hsp_ref, dinv_ref, b_ref, wh_ref, bh_ref, out_ref):
        pre = (a0_ref[0] + a1_ref[0] + hsp_ref[...]) * dinv_ref[...] + b_ref[...]
        h = jnp.maximum(pre, 0.0)
        out_ref[...] = (
            jnp.dot(h, wh_ref[...], preferred_element_type=jnp.float32) + bh_ref[...]
        )

    return pl.pallas_call(
        body,
        grid=(N_NODES // _R,),
        in_specs=[
            pl.BlockSpec((1, _R, D), lambda i: (0, i, 0)),
            pl.BlockSpec((1, _R, D), lambda i: (1, i, 0)),
            pl.BlockSpec((_R, D), lambda i: (i, 0)),
            pl.BlockSpec((_R, 1), lambda i: (i, 0)),
            pl.BlockSpec((1, D), lambda i: (0, 0)),
            pl.BlockSpec((D, OUT_DIM), lambda i: (0, 0)),
            pl.BlockSpec((1, OUT_DIM), lambda i: (0, 0)),
        ],
        out_specs=pl.BlockSpec((_R, OUT_DIM), lambda i: (i, 0)),
        out_shape=jax.ShapeDtypeStruct((N_NODES, OUT_DIM), jnp.float32),
    )(agg, agg, hs_prev, dinv, b, Wh, bh)


def kernel(x, edge_index, W1, b1, W2, b2, Wh, bh):
    ei = edge_index.astype(jnp.int32).reshape(2, NW, NWIN, WIN, GCHUNK)
    zrows = jnp.zeros((N_PAD, D), jnp.float32)

    hist = _sc_hist(ei[1].reshape(NW, NCHUNK, CHUNK))
    h1 = _tc_matmul(x, W1)

    hs1, dinv = _tc_scale(h1, hist.reshape(NC, N_PAD, 1))
    agg1 = _sc_scatter(hs1, ei, zrows)
    hs2 = _tc_mid(agg1, hs1, dinv, b1.reshape(1, D), W2)
    agg2 = _sc_scatter(hs2, ei, zrows)
    z = _tc_last(agg2, hs2, dinv, b2.reshape(1, D), Wh, bh.reshape(1, OUT_DIM))
    return z


# speedup vs baseline: 1.0489x; 1.0162x over previous
import functools

import jax
import jax.numpy as jnp
from jax import lax
from jax.experimental import pallas as pl
from jax.experimental.pallas import tpu as pltpu
from jax.experimental.pallas import tpu_sc as plsc

N_NODES = 10000
N_EDGES = 320000
D = 128
OUT_DIM = 64

NC, NS = 2, 16
NW = NC * NS
E_PER_W = N_EDGES // NW
CHUNK = 80
NCHUNK = E_PER_W // CHUNK
N_PAD = 10240
H_PER_TILE = N_PAD // NS

GCHUNK = 80
GNCHUNK = 125
WIN = 5
NWIN = GNCHUNK // WIN


def _mesh():
    return plsc.VectorSubcoreMesh(
        core_axis_name="c", subcore_axis_name="s", num_cores=NC, num_subcores=NS
    )


def _sc_hist(ei):
    @functools.partial(
        pl.kernel,
        mesh=_mesh(),
        out_type=jax.ShapeDtypeStruct((NC, 1, N_PAD), jnp.float32),
        scratch_types=[
            pltpu.VMEM((NWIN, WIN, GCHUNK), jnp.int32),
            pltpu.VMEM((GCHUNK,), jnp.float32),
            pltpu.VMEM((H_PER_TILE,), jnp.float32),
            pltpu.VMEM_SHARED((N_PAD,), jnp.float32),
        ],
    )
    def k(ei_hbm, out_hbm, dst_v, ones_v, zbuf_v, hist_sh):
        c = lax.axis_index("c")
        s = lax.axis_index("s")
        w = c * NS + s
        zeros16 = jnp.zeros((16,), jnp.float32)
        ones16 = jnp.ones((16,), jnp.float32)

        def zero_body(i, carry):
            zbuf_v[pl.ds(i * 16, 16)] = zeros16
            return carry

        lax.fori_loop(0, H_PER_TILE // 16, zero_body, 0)
        for q in range(GCHUNK // 16):
            ones_v[pl.ds(q * 16, 16)] = ones16
        pltpu.sync_copy(zbuf_v, hist_sh.at[pl.ds(s * H_PER_TILE, H_PER_TILE)])
        pltpu.sync_copy(ei_hbm.at[1, w], dst_v)
        plsc.subcore_barrier()

        def body(o, carry):
            for j in range(WIN):
                pltpu.sync_copy(ones_v, hist_sh.at[dst_v.at[o, j]], add=True)
            return carry

        lax.fori_loop(0, NWIN, body, 0)
        plsc.subcore_barrier()
        pltpu.sync_copy(
            hist_sh.at[pl.ds(s * H_PER_TILE, H_PER_TILE)],
            out_hbm.at[c, 0, pl.ds(s * H_PER_TILE, H_PER_TILE)],
        )

    return k(ei)


def _sc_scatter(hs, ei, zrows):
    @functools.partial(
        pl.kernel,
        mesh=_mesh(),
        out_type=jax.ShapeDtypeStruct((NC, N_PAD, D), jnp.float32),
        scratch_types=[
            pltpu.VMEM((3, WIN, GCHUNK), jnp.int32),
            pltpu.VMEM((3, WIN, GCHUNK), jnp.int32),
            pltpu.VMEM((4, GCHUNK, D), jnp.float32),
            pltpu.VMEM_SHARED((N_PAD, D), jnp.float32),
            pltpu.SemaphoreType.DMA((4,)),
            pltpu.SemaphoreType.DMA((4,)),
            pltpu.SemaphoreType.DMA((3,)),
        ],
    )
    def k(hs_hbm, ei_hbm, z_hbm, out_hbm, src_v, dst_v, rows_v, acc_sh, gsem, ssem, isem):
        src_hbm = ei_hbm.at[0]
        dst_hbm = ei_hbm.at[1]
        c = lax.axis_index("c")
        s = lax.axis_index("s")
        w = c * NS + s
        zcopy = pltpu.async_copy(
            z_hbm.at[pl.ds(s * H_PER_TILE, H_PER_TILE)],
            acc_sh.at[pl.ds(s * H_PER_TILE, H_PER_TILE)],
            gsem.at[2],
        )
        pltpu.sync_copy(src_hbm.at[w, 0], src_v.at[0])
        pltpu.sync_copy(dst_hbm.at[w, 0], dst_v.at[0])
        pltpu.async_copy(src_hbm.at[w, 1], src_v.at[1], isem.at[1])
        pltpu.async_copy(dst_hbm.at[w, 1], dst_v.at[1], isem.at[1])
        zcopy.wait()
        plsc.subcore_barrier()

        pltpu.async_copy(hs_hbm.at[src_v.at[0, 0]], rows_v.at[0], gsem.at[0])
        pltpu.async_copy(hs_hbm.at[src_v.at[0, 1]], rows_v.at[1], gsem.at[1])

        def outer(o, carry):
            om = lax.rem(o, 3)
            om1 = lax.rem(o + 1, 3)
            om2 = lax.rem(o + 2, 3)
            for j in range(WIN):
                cidx = o * WIN + j
                b = lax.rem(cidx, 4)

                if j == 3:
                    @pl.when(o + 1 < NWIN)
                    def _():
                        pltpu.make_async_copy(
                            src_hbm.at[w, o + 1], src_v.at[om1], isem.at[om1]
                        ).wait()
                        pltpu.make_async_copy(
                            dst_hbm.at[w, o + 1], dst_v.at[om1], isem.at[om1]
                        ).wait()

                g = cidx + 2
                bg = lax.rem(g, 4)
                gidx = src_v.at[om, j + 2] if j < 3 else src_v.at[om1, j - 3]

                @pl.when(g < GNCHUNK)
                def _():
                    @pl.when(cidx >= 2)
                    def _():
                        pltpu.make_async_copy(
                            rows_v.at[bg], acc_sh.at[dst_v.at[om, 0]], ssem.at[bg]
                        ).wait()

                    pltpu.async_copy(hs_hbm.at[gidx], rows_v.at[bg], gsem.at[bg])

                pltpu.make_async_copy(
                    hs_hbm.at[src_v.at[om, j]], rows_v.at[b], gsem.at[b]
                ).wait()
                pltpu.async_copy(
                    rows_v.at[b], acc_sh.at[dst_v.at[om, j]], ssem.at[b], add=True
                )

                if j == 4:

                    @pl.when(o + 2 < NWIN)
                    def _():
                        pltpu.async_copy(src_hbm.at[w, o + 2], src_v.at[om2], isem.at[om2])
                        pltpu.async_copy(dst_hbm.at[w, o + 2], dst_v.at[om2], isem.at[om2])

            return carry

        lax.fori_loop(0, NWIN, outer, 0)
        for b in range(4):
            pltpu.make_async_copy(
                rows_v.at[b], acc_sh.at[dst_v.at[0, 0]], ssem.at[b]
            ).wait()
        plsc.subcore_barrier()
        pltpu.sync_copy(
            acc_sh.at[pl.ds(s * H_PER_TILE, H_PER_TILE)],
            out_hbm.at[c, pl.ds(s * H_PER_TILE, H_PER_TILE)],
        )

    return k(hs, ei, zrows)


_R = 2000


def _tc_matmul(x, W1):
    def body(x_ref, w_ref, out_ref):
        out_ref[...] = jnp.dot(x_ref[...], w_ref[...], preferred_element_type=jnp.float32)

    return pl.pallas_call(
        body,
        grid=(N_NODES // _R,),
        in_specs=[
            pl.BlockSpec((_R, D), lambda i: (i, 0)),
            pl.BlockSpec((D, D), lambda i: (0, 0)),
        ],
        out_specs=pl.BlockSpec((_R, D), lambda i: (i, 0)),
        out_shape=jax.ShapeDtypeStruct((N_NODES, D), jnp.float32),
    )(x, W1)


def _tc_scale(h1, hist):
    def body(h_ref, h0_ref, h1_ref, hs_ref, dinv_ref):
        deg = h0_ref[0] + h1_ref[0] + 1.0
        dinv = lax.rsqrt(deg)
        hs_ref[...] = h_ref[...] * dinv
        dinv_ref[...] = dinv

    return pl.pallas_call(
        body,
        grid=(N_NODES // _R,),
        in_specs=[
            pl.BlockSpec((_R, D), lambda i: (i, 0)),
            pl.BlockSpec((1, _R, 1), lambda i: (0, i, 0)),
            pl.BlockSpec((1, _R, 1), lambda i: (1, i, 0)),
        ],
        out_specs=[
            pl.BlockSpec((_R, D), lambda i: (i, 0)),
            pl.BlockSpec((_R, 1), lambda i: (i, 0)),
        ],
        out_shape=[
            jax.ShapeDtypeStruct((N_NODES, D), jnp.float32),
            jax.ShapeDtypeStruct((N_NODES, 1), jnp.float32),
        ],
    )(h1, hist, hist)


def _tc_mid(agg, hs_prev, dinv, b, W):
    def body(a0_ref, a1_ref, hsp_ref, dinv_ref, b_ref, w_ref, out_ref):
        dinv = dinv_ref[...]
        pre = (a0_ref[0] + a1_ref[0] + hsp_ref[...]) * dinv + b_ref[...]
        h = jnp.maximum(pre, 0.0)
        out_ref[...] = jnp.dot(h, w_ref[...], preferred_element_type=jnp.float32) * dinv

    return pl.pallas_call(
        body,
        grid=(N_NODES // _R,),
        in_specs=[
            pl.BlockSpec((1, _R, D), lambda i: (0, i, 0)),
            pl.BlockSpec((1, _R, D), lambda i: (1, i, 0)),
            pl.BlockSpec((_R, D), lambda i: (i, 0)),
            pl.BlockSpec((_R, 1), lambda i: (i, 0)),
            pl.BlockSpec((1, D), lambda i: (0, 0)),
            pl.BlockSpec((D, D), lambda i: (0, 0)),
        ],
        out_specs=pl.BlockSpec((_R, D), lambda i: (i, 0)),
        out_shape=jax.ShapeDtypeStruct((N_NODES, D), jnp.float32),
    )(agg, agg, hs_prev, dinv, b, W)


def _tc_last(agg, hs_prev, dinv, b, Wh, bh):
    def body(a0_ref, a1_ref, hsp_ref, dinv_ref, b_ref, wh_ref, bh_ref, out_ref):
        pre = (a0_ref[0] + a1_ref[0] + hsp_ref[...]) * dinv_ref[...] + b_ref[...]
        h = jnp.maximum(pre, 0.0)
        out_ref[...] = (
            jnp.dot(h, wh_ref[...], preferred_element_type=jnp.float32) + bh_ref[...]
        )

    return pl.pallas_call(
        body,
        grid=(N_NODES // _R,),
        in_specs=[
            pl.BlockSpec((1, _R, D), lambda i: (0, i, 0)),
            pl.BlockSpec((1, _R, D), lambda i: (1, i, 0)),
            pl.BlockSpec((_R, D), lambda i: (i, 0)),
            pl.BlockSpec((_R, 1), lambda i: (i, 0)),
            pl.BlockSpec((1, D), lambda i: (0, 0)),
            pl.BlockSpec((D, OUT_DIM), lambda i: (0, 0)),
            pl.BlockSpec((1, OUT_DIM), lambda i: (0, 0)),
        ],
        out_specs=pl.BlockSpec((_R, OUT_DIM), lambda i: (i, 0)),
        out_shape=jax.ShapeDtypeStruct((N_NODES, OUT_DIM), jnp.float32),
    )(agg, agg, hs_prev, dinv, b, Wh, bh)


def kernel(x, edge_index, W1, b1, W2, b2, Wh, bh):
    ei = edge_index.astype(jnp.int32).reshape(2, NW, NWIN, WIN, GCHUNK)
    zrows = jnp.zeros((N_PAD, D), jnp.float32)

    hist = _sc_hist(ei)
    h1 = _tc_matmul(x, W1)

    hs1, dinv = _tc_scale(h1, hist.reshape(NC, N_PAD, 1))
    agg1 = _sc_scatter(hs1, ei, zrows)
    hs2 = _tc_mid(agg1, hs1, dinv, b1.reshape(1, D), W2)
    agg2 = _sc_scatter(hs2, ei, zrows)
    z = _tc_last(agg2, hs2, dinv, b2.reshape(1, D), Wh, bh.reshape(1, OUT_DIM))
    return z


# speedup vs baseline: 1.0500x; 1.0010x over previous
import functools

import jax
import jax.numpy as jnp
from jax import lax
from jax.experimental import pallas as pl
from jax.experimental.pallas import tpu as pltpu
from jax.experimental.pallas import tpu_sc as plsc

N_NODES = 10000
N_EDGES = 320000
D = 128
OUT_DIM = 64

NC, NS = 2, 16
NW = NC * NS
E_PER_W = N_EDGES // NW
N_PAD = 10240
H_PER_TILE = N_PAD // NS

GCHUNK = 80
GNCHUNK = 125
WIN = 5
NWIN = GNCHUNK // WIN


def _mesh():
    return plsc.VectorSubcoreMesh(
        core_axis_name="c", subcore_axis_name="s", num_cores=NC, num_subcores=NS
    )


def _sc_hist(ei):
    @functools.partial(
        pl.kernel,
        mesh=_mesh(),
        out_type=jax.ShapeDtypeStruct((NC, 1, N_PAD), jnp.float32),
        scratch_types=[
            pltpu.VMEM((NWIN, WIN, GCHUNK), jnp.int32),
            pltpu.VMEM((GCHUNK,), jnp.float32),
            pltpu.VMEM((H_PER_TILE,), jnp.float32),
            pltpu.VMEM_SHARED((N_PAD,), jnp.float32),
        ],
    )
    def k(ei_hbm, out_hbm, dst_v, ones_v, zbuf_v, hist_sh):
        c = lax.axis_index("c")
        s = lax.axis_index("s")
        w = c * NS + s
        zeros16 = jnp.zeros((16,), jnp.float32)
        ones16 = jnp.ones((16,), jnp.float32)

        def zero_body(i, carry):
            zbuf_v[pl.ds(i * 16, 16)] = zeros16
            return carry

        lax.fori_loop(0, H_PER_TILE // 16, zero_body, 0)
        for q in range(GCHUNK // 16):
            ones_v[pl.ds(q * 16, 16)] = ones16
        pltpu.sync_copy(zbuf_v, hist_sh.at[pl.ds(s * H_PER_TILE, H_PER_TILE)])
        pltpu.sync_copy(ei_hbm.at[1, w], dst_v)
        plsc.subcore_barrier()

        def body(o, carry):
            for j in range(WIN):
                pltpu.sync_copy(ones_v, hist_sh.at[dst_v.at[o, j]], add=True)
            return carry

        lax.fori_loop(0, NWIN, body, 0)
        plsc.subcore_barrier()
        pltpu.sync_copy(
            hist_sh.at[pl.ds(s * H_PER_TILE, H_PER_TILE)],
            out_hbm.at[c, 0, pl.ds(s * H_PER_TILE, H_PER_TILE)],
        )

    return k(ei)


def _sc_scatter(hs, ei, zrows):
    @functools.partial(
        pl.kernel,
        mesh=_mesh(),
        out_type=jax.ShapeDtypeStruct((NC, N_PAD, D), jnp.float32),
        scratch_types=[
            pltpu.VMEM((3, WIN, GCHUNK), jnp.int32),
            pltpu.VMEM((3, WIN, GCHUNK), jnp.int32),
            pltpu.VMEM((4, GCHUNK, D), jnp.float32),
            pltpu.VMEM_SHARED((N_PAD, D), jnp.float32),
            pltpu.SemaphoreType.DMA((4,)),
            pltpu.SemaphoreType.DMA((4,)),
            pltpu.SemaphoreType.DMA((3,)),
        ],
    )
    def k(hs_hbm, ei_hbm, z_hbm, out_hbm, src_v, dst_v, rows_v, acc_sh, gsem, ssem, isem):
        src_hbm = ei_hbm.at[0]
        dst_hbm = ei_hbm.at[1]
        c = lax.axis_index("c")
        s = lax.axis_index("s")
        w = c * NS + s
        zcopy = pltpu.async_copy(
            z_hbm.at[pl.ds(s * H_PER_TILE, H_PER_TILE)],
            acc_sh.at[pl.ds(s * H_PER_TILE, H_PER_TILE)],
            gsem.at[2],
        )
        pltpu.sync_copy(src_hbm.at[w, 0], src_v.at[0])
        pltpu.sync_copy(dst_hbm.at[w, 0], dst_v.at[0])
        pltpu.async_copy(src_hbm.at[w, 1], src_v.at[1], isem.at[1])
        pltpu.async_copy(dst_hbm.at[w, 1], dst_v.at[1], isem.at[1])
        zcopy.wait()
        plsc.subcore_barrier()

        pltpu.async_copy(hs_hbm.at[src_v.at[0, 0]], rows_v.at[0], gsem.at[0])
        pltpu.async_copy(hs_hbm.at[src_v.at[0, 1]], rows_v.at[1], gsem.at[1])

        def outer(o, carry):
            om = lax.rem(o, 3)
            om1 = lax.rem(o + 1, 3)
            om2 = lax.rem(o + 2, 3)
            for j in range(WIN):
                cidx = o * WIN + j
                b = lax.rem(cidx, 4)

                if j == 3:
                    @pl.when(o + 1 < NWIN)
                    def _():
                        pltpu.make_async_copy(
                            src_hbm.at[w, o + 1], src_v.at[om1], isem.at[om1]
                        ).wait()
                        pltpu.make_async_copy(
                            dst_hbm.at[w, o + 1], dst_v.at[om1], isem.at[om1]
                        ).wait()

                g = cidx + 2
                bg = lax.rem(g, 4)
                gidx = src_v.at[om, j + 2] if j < 3 else src_v.at[om1, j - 3]

                @pl.when(g < GNCHUNK)
                def _():
                    @pl.when(cidx >= 2)
                    def _():
                        pltpu.make_async_copy(
                            rows_v.at[bg], acc_sh.at[dst_v.at[om, 0]], ssem.at[bg]
                        ).wait()

                    pltpu.async_copy(hs_hbm.at[gidx], rows_v.at[bg], gsem.at[bg])

                pltpu.make_async_copy(
                    hs_hbm.at[src_v.at[om, j]], rows_v.at[b], gsem.at[b]
                ).wait()
                pltpu.async_copy(
                    rows_v.at[b], acc_sh.at[dst_v.at[om, j]], ssem.at[b], add=True
                )

                if j == 4:

                    @pl.when(o + 2 < NWIN)
                    def _():
                        pltpu.async_copy(src_hbm.at[w, o + 2], src_v.at[om2], isem.at[om2])
                        pltpu.async_copy(dst_hbm.at[w, o + 2], dst_v.at[om2], isem.at[om2])

            return carry

        lax.fori_loop(0, NWIN, outer, 0)
        for b in range(4):
            pltpu.make_async_copy(
                rows_v.at[b], acc_sh.at[dst_v.at[0, 0]], ssem.at[b]
            ).wait()
        plsc.subcore_barrier()
        pltpu.sync_copy(
            acc_sh.at[pl.ds(s * H_PER_TILE, H_PER_TILE)],
            out_hbm.at[c, pl.ds(s * H_PER_TILE, H_PER_TILE)],
        )

    return k(hs, ei, zrows)


_R = 2000


def _tc_matmul(x, W1):
    def body(x_ref, w_ref, out_ref):
        out_ref[...] = jnp.dot(x_ref[...], w_ref[...], preferred_element_type=jnp.float32)

    return pl.pallas_call(
        body,
        grid=(N_NODES // _R,),
        in_specs=[
            pl.BlockSpec((_R, D), lambda i: (i, 0)),
            pl.BlockSpec((D, D), lambda i: (0, 0)),
        ],
        out_specs=pl.BlockSpec((_R, D), lambda i: (i, 0)),
        out_shape=jax.ShapeDtypeStruct((N_NODES, D), jnp.float32),
    )(x, W1)


def _tc_scale(h1, hist):
    def body(h_ref, h0_ref, h1_ref, hs_ref, dinv_ref):
        deg = h0_ref[0] + h1_ref[0] + 1.0
        dinv = lax.rsqrt(deg)
        hs_ref[...] = h_ref[...] * dinv
        dinv_ref[...] = dinv

    return pl.pallas_call(
        body,
        grid=(N_NODES // _R,),
        in_specs=[
            pl.BlockSpec((_R, D), lambda i: (i, 0)),
            pl.BlockSpec((1, _R, 1), lambda i: (0, i, 0)),
            pl.BlockSpec((1, _R, 1), lambda i: (1, i, 0)),
        ],
        out_specs=[
            pl.BlockSpec((_R, D), lambda i: (i, 0)),
            pl.BlockSpec((_R, 1), lambda i: (i, 0)),
        ],
        out_shape=[
            jax.ShapeDtypeStruct((N_NODES, D), jnp.float32),
            jax.ShapeDtypeStruct((N_NODES, 1), jnp.float32),
        ],
    )(h1, hist, hist)


def _tc_mid(agg, hs_prev, dinv, b, W):
    def body(a0_ref, a1_ref, hsp_ref, dinv_ref, b_ref, w_ref, out_ref):
        dinv = dinv_ref[...]
        pre = (a0_ref[0] + a1_ref[0] + hsp_ref[...]) * dinv + b_ref[...]
        h = jnp.maximum(pre, 0.0)
        out_ref[...] = jnp.dot(h, w_ref[...], preferred_element_type=jnp.float32) * dinv

    return pl.pallas_call(
        body,
        grid=(N_NODES // _R,),
        in_specs=[
            pl.BlockSpec((1, _R, D), lambda i: (0, i, 0)),
            pl.BlockSpec((1, _R, D), lambda i: (1, i, 0)),
            pl.BlockSpec((_R, D), lambda i: (i, 0)),
            pl.BlockSpec((_R, 1), lambda i: (i, 0)),
            pl.BlockSpec((1, D), lambda i: (0, 0)),
            pl.BlockSpec((D, D), lambda i: (0, 0)),
        ],
        out_specs=pl.BlockSpec((_R, D), lambda i: (i, 0)),
        out_shape=jax.ShapeDtypeStruct((N_NODES, D), jnp.float32),
    )(agg, agg, hs_prev, dinv, b, W)


def _tc_last(agg, hs_prev, dinv, b, Wh, bh):
    def body(a0_ref, a1_ref, hsp_ref, dinv_ref, b_ref, wh_ref, bh_ref, out_ref):
        pre = (a0_ref[0] + a1_ref[0] + hsp_ref[...]) * dinv_ref[...] + b_ref[...]
        h = jnp.maximum(pre, 0.0)
        out_ref[...] = (
            jnp.dot(h, wh_ref[...], preferred_element_type=jnp.float32) + bh_ref[...]
        )

    return pl.pallas_call(
        body,
        grid=(N_NODES // _R,),
        in_specs=[
            pl.BlockSpec((1, _R, D), lambda i: (0, i, 0)),
            pl.BlockSpec((1, _R, D), lambda i: (1, i, 0)),
            pl.BlockSpec((_R, D), lambda i: (i, 0)),
            pl.BlockSpec((_R, 1), lambda i: (i, 0)),
            pl.BlockSpec((1, D), lambda i: (0, 0)),
            pl.BlockSpec((D, OUT_DIM), lambda i: (0, 0)),
            pl.BlockSpec((1, OUT_DIM), lambda i: (0, 0)),
        ],
        out_specs=pl.BlockSpec((_R, OUT_DIM), lambda i: (i, 0)),
        out_shape=jax.ShapeDtypeStruct((N_NODES, OUT_DIM), jnp.float32),
    )(agg, agg, hs_prev, dinv, b, Wh, bh)


def kernel(x, edge_index, W1, b1, W2, b2, Wh, bh):
    ei = edge_index.astype(jnp.int32).reshape(2, NW, NWIN, WIN, GCHUNK)
    zrows = jnp.zeros((N_PAD, D), jnp.float32)

    hist = _sc_hist(ei)
    h1 = _tc_matmul(x, W1)

    hs1, dinv = _tc_scale(h1, hist.reshape(NC, N_PAD, 1))
    agg1 = _sc_scatter(hs1, ei, zrows)
    hs2 = _tc_mid(agg1, hs1, dinv, b1.reshape(1, D), W2)
    agg2 = _sc_scatter(hs2, ei, zrows)
    z = _tc_last(agg2, hs2, dinv, b2.reshape(1, D), Wh, bh.reshape(1, OUT_DIM))
    return z
